# baseline (speedup 1.0000x reference)
import functools

import jax
import jax.numpy as jnp
from jax import lax
from jax.experimental import pallas as pl
from jax.experimental.pallas import tpu as pltpu, tpu_sc as plsc

_LANE = 128
_NBUF = 16


def kernel(indices_or_features, embedding_weight):
    idx = jnp.squeeze(indices_or_features).astype(jnp.int32)
    (B,) = idx.shape
    V, D = embedding_weight.shape
    tbl_t = embedding_weight.T

    info = plsc.get_sparse_core_info()
    NC, NS = info.num_cores, info.num_subcores
    NW = NC * NS
    assert B % (8 * NW) == 0 and D == 32
    b_per_w = B // NW
    assert b_per_w % _NBUF == 0

    mesh = plsc.VectorSubcoreMesh(core_axis_name="c", subcore_axis_name="s")

    @functools.partial(
        pl.kernel,
        mesh=mesh,
        out_type=jax.ShapeDtypeStruct((D, B), jnp.float32),
        compiler_params=pltpu.CompilerParams(
            use_tc_tiling_on_sc=True, needs_layout_passes=False
        ),
        scratch_types=[
            pltpu.VMEM((b_per_w + 16,), jnp.int32),
            pltpu.VMEM((_NBUF, D, _LANE), jnp.float32),
            pltpu.VMEM((D, b_per_w), jnp.float32),
            pltpu.SemaphoreType.DMA,
        ]
        + [pltpu.SemaphoreType.DMA] * _NBUF,
    )
    def gather_kernel(idx_hbm, tbl_hbm, out_hbm, idx_v, ring_v, stage_v, sem_i, *sems):
        wid = lax.axis_index("s") * NC + lax.axis_index("c")
        base = wid * b_per_w
        pltpu.async_copy(
            idx_hbm.at[pl.ds(base, b_per_w)], idx_v.at[pl.ds(0, b_per_w)], sem_i
        ).wait()

        rows = lax.iota(jnp.int32, 16)

        def load_idx(j):
            return idx_v[pl.ds(j, 16)][0]

        def fetch(j, s):
            i = load_idx(j)
            col0 = pl.multiple_of((i // _LANE) * _LANE, _LANE)
            pltpu.async_copy(
                tbl_hbm.at[:, pl.ds(col0, _LANE)],
                ring_v.at[s],
                sems[s],
            )

        def extract(j, s):
            pltpu.make_async_copy(
                tbl_hbm.at[:, pl.ds(0, _LANE)], ring_v.at[s], sems[s]
            ).wait()
            i = load_idx(j)
            slot = jnp.full((16,), s, jnp.int32)
            col = jnp.full((16,), i % _LANE, jnp.int32)
            jcol = jnp.full((16,), j, jnp.int32)
            for h in range(2):
                vals = plsc.load_gather(ring_v, [slot, rows + h * 16, col])
                plsc.store_scatter(stage_v, [rows + h * 16, jcol], vals)

        for s in range(_NBUF):
            fetch(s, s)

        def body(g, carry):
            j0 = g * _NBUF
            for s in range(_NBUF):
                extract(j0 + s, s)
                fetch(j0 + s + _NBUF, s)
            return carry

        lax.fori_loop(0, b_per_w // _NBUF - 1, body, 0)

        for s in range(_NBUF):
            extract(b_per_w - _NBUF + s, s)

        pltpu.sync_copy(stage_v, out_hbm.at[:, pl.ds(base, b_per_w)])

    return gather_kernel(idx, tbl_t).T


# speedup vs baseline: 1.0126x; 1.0126x over previous
import functools

import jax
import jax.numpy as jnp
from jax import lax
from jax.experimental import pallas as pl
from jax.experimental.pallas import tpu as pltpu, tpu_sc as plsc

_LANE = 128
_NBUF = 8


def kernel(indices_or_features, embedding_weight):
    idx = jnp.squeeze(indices_or_features).astype(jnp.int32)
    (B,) = idx.shape
    V, D = embedding_weight.shape
    tbl_t = embedding_weight.T

    info = plsc.get_sparse_core_info()
    NC, NS = info.num_cores, info.num_subcores
    NW = NC * NS
    assert B % (8 * NW) == 0 and D == 32
    b_per_w = B // NW
    assert b_per_w % _NBUF == 0

    mesh = plsc.VectorSubcoreMesh(core_axis_name="c", subcore_axis_name="s")

    @functools.partial(
        pl.kernel,
        mesh=mesh,
        out_type=jax.ShapeDtypeStruct((D, B), jnp.float32),
        compiler_params=pltpu.CompilerParams(
            use_tc_tiling_on_sc=True, needs_layout_passes=False
        ),
        scratch_types=[
            pltpu.VMEM((b_per_w + 16,), jnp.int32),
            pltpu.VMEM((_NBUF, D, _LANE), jnp.float32),
            pltpu.VMEM((D, b_per_w), jnp.float32),
            pltpu.SemaphoreType.DMA,
        ]
        + [pltpu.SemaphoreType.DMA] * _NBUF,
    )
    def gather_kernel(idx_hbm, tbl_hbm, out_hbm, idx_v, ring_v, stage_v, sem_i, *sems):
        wid = lax.axis_index("s") * NC + lax.axis_index("c")
        base = wid * b_per_w
        pltpu.async_copy(
            idx_hbm.at[pl.ds(base, b_per_w)], idx_v.at[pl.ds(0, b_per_w)], sem_i
        ).wait()

        rows = lax.iota(jnp.int32, 16)

        def load_idx(j):
            return idx_v[pl.ds(j, 16)][0]

        def fetch(j, s):
            i = load_idx(j)
            col0 = pl.multiple_of((i // _LANE) * _LANE, _LANE)
            pltpu.async_copy(
                tbl_hbm.at[:, pl.ds(col0, _LANE)],
                ring_v.at[s],
                sems[s],
            )

        def extract(j, s):
            pltpu.make_async_copy(
                tbl_hbm.at[:, pl.ds(0, _LANE)], ring_v.at[s], sems[s]
            ).wait()
            i = load_idx(j)
            slot = jnp.full((16,), s, jnp.int32)
            col = jnp.full((16,), i % _LANE, jnp.int32)
            jcol = jnp.full((16,), j, jnp.int32)
            for h in range(2):
                vals = plsc.load_gather(ring_v, [slot, rows + h * 16, col])
                plsc.store_scatter(stage_v, [rows + h * 16, jcol], vals)

        for s in range(_NBUF):
            fetch(s, s)

        def body(g, carry):
            j0 = g * _NBUF
            for s in range(_NBUF):
                extract(j0 + s, s)
                fetch(j0 + s + _NBUF, s)
            return carry

        lax.fori_loop(0, b_per_w // _NBUF - 1, body, 0)

        for s in range(_NBUF):
            extract(b_per_w - _NBUF + s, s)

        pltpu.sync_copy(stage_v, out_hbm.at[:, pl.ds(base, b_per_w)])

    return gather_kernel(idx, tbl_t).T


# speedup vs baseline: 1.0226x; 1.0099x over previous
import functools

import jax
import jax.numpy as jnp
from jax import lax
from jax.experimental import pallas as pl
from jax.experimental.pallas import tpu as pltpu, tpu_sc as plsc

_LANE = 128
_NBUF = 8


def kernel(indices_or_features, embedding_weight):
    idx = jnp.squeeze(indices_or_features).astype(jnp.int32)
    (B,) = idx.shape
    V, D = embedding_weight.shape
    tbl_t = embedding_weight.T

    info = plsc.get_sparse_core_info()
    NC, NS = info.num_cores, info.num_subcores
    NW = NC * NS
    assert B % (8 * NW) == 0 and D == 32
    b_per_w = B // NW
    assert b_per_w % _NBUF == 0

    mesh = plsc.VectorSubcoreMesh(core_axis_name="c", subcore_axis_name="s")

    @functools.partial(
        pl.kernel,
        mesh=mesh,
        out_type=jax.ShapeDtypeStruct((D, B), jnp.float32),
        compiler_params=pltpu.CompilerParams(
            use_tc_tiling_on_sc=True, needs_layout_passes=False
        ),
        scratch_types=[
            pltpu.VMEM((b_per_w + 16,), jnp.int32),
            pltpu.VMEM((_NBUF, D, _LANE), jnp.float32),
            pltpu.VMEM((D, b_per_w), jnp.float32),
            pltpu.SemaphoreType.DMA,
        ]
        + [pltpu.SemaphoreType.DMA] * _NBUF,
    )
    def gather_kernel(idx_hbm, tbl_hbm, out_hbm, idx_v, ring_v, stage_v, sem_i, *sems):
        wid = lax.axis_index("s") * NC + lax.axis_index("c")
        base = wid * b_per_w
        pltpu.async_copy(
            idx_hbm.at[pl.ds(base, b_per_w)], idx_v.at[pl.ds(0, b_per_w)], sem_i
        ).wait()

        rows = lax.iota(jnp.int32, 16)

        def load_idx(j):
            return idx_v[pl.ds(j, 16)][0]

        def fetch(j, s):
            i = load_idx(j)
            col0 = pl.multiple_of((i // _LANE) * _LANE, _LANE)
            for g in range(D // 8):
                pltpu.async_copy(
                    tbl_hbm.at[pl.ds(8 * g, 8), pl.ds(col0, _LANE)],
                    ring_v.at[s, pl.ds(8 * g, 8), :],
                    sems[s],
                )

        def extract(j, s):
            pltpu.make_async_copy(
                tbl_hbm.at[:, pl.ds(0, _LANE)], ring_v.at[s], sems[s]
            ).wait()
            i = load_idx(j)
            slot = jnp.full((16,), s, jnp.int32)
            col = jnp.full((16,), i % _LANE, jnp.int32)
            jcol = jnp.full((16,), j, jnp.int32)
            for h in range(2):
                vals = plsc.load_gather(ring_v, [slot, rows + h * 16, col])
                plsc.store_scatter(stage_v, [rows + h * 16, jcol], vals)

        for s in range(_NBUF):
            fetch(s, s)

        def body(g, carry):
            j0 = g * _NBUF
            for s in range(_NBUF):
                extract(j0 + s, s)
                fetch(j0 + s + _NBUF, s)
            return carry

        lax.fori_loop(0, b_per_w // _NBUF - 1, body, 0)

        for s in range(_NBUF):
            extract(b_per_w - _NBUF + s, s)

        pltpu.sync_copy(stage_v, out_hbm.at[:, pl.ds(base, b_per_w)])

    return gather_kernel(idx, tbl_t).T
